# baseline (speedup 1.0000x reference)
import functools

import jax
import jax.numpy as jnp
from jax import lax
from jax.experimental import pallas as pl
from jax.experimental.pallas import tpu as pltpu
from jax.experimental.pallas import tpu_sc as plsc

_B, _S, _D, _V = 128, 77, 256, 50000
_SP = 80
_G = 8
_ROWS = _G * _SP
_NQ = 1
_BQ = _B // _NQ
_NROWS_Q = _BQ * _SP


def _sc_gather(table, idx_q):
    info = plsc.get_sparse_core_info()
    nw = info.num_cores * info.num_subcores
    b_per_w = _NROWS_Q // nw
    n_streams = b_per_w // _SP

    mesh = plsc.VectorSubcoreMesh(core_axis_name="c", subcore_axis_name="s")

    @functools.partial(
        pl.kernel,
        out_type=jax.ShapeDtypeStruct((_NROWS_Q, _D), jnp.float32),
        mesh=mesh,
        scratch_types=[
            pltpu.VMEM((n_streams, _SP), jnp.int32),
            pltpu.VMEM((b_per_w, _D), jnp.float32),
            [pltpu.SemaphoreType.DMA] * 4,
            [pltpu.SemaphoreType.DMA] * 4,
        ],
    )
    def k(table_hbm, idx_hbm, out_hbm, idx_v, rows_v, gsems, wsems):
        wid = lax.axis_index("s") * info.num_cores + lax.axis_index("c")
        base = wid * b_per_w
        pltpu.sync_copy(idx_hbm.at[wid], idx_v)
        gathers = []
        for c in range(n_streams):
            gathers.append(pltpu.async_copy(
                table_hbm.at[idx_v.at[c]],
                rows_v.at[pl.ds(c * _SP, _SP)], gsems[c % 4]))
        writes = []
        for c in range(n_streams):
            gathers[c].wait()
            writes.append(pltpu.async_copy(
                rows_v.at[pl.ds(c * _SP, _SP)],
                out_hbm.at[pl.ds(base + c * _SP, _SP)], wsems[c % 4]))
        for wcp in writes:
            wcp.wait()

    return k(table, idx_q.reshape(nw, n_streams, _SP))


def _cap(u):
    n = jnp.maximum(jnp.sqrt(jnp.sum(u * u, axis=-1, keepdims=True)), 1e-15)
    thc = jnp.minimum(jnp.tanh(n), 1.0 - 1e-7)
    s = 0.5 * jnp.log((1.0 + thc) / (1.0 - thc)) / n
    return u * s


def _nt(a, w):
    return lax.dot_general(a.astype(jnp.bfloat16), w.astype(jnp.bfloat16),
                           (((1,), (1,)), ((), ())),
                           preferred_element_type=jnp.float32)


def _tc_body(x_ref, edge_ref, pos_ref,
             w0, w1, w2, w3, w4, pb_ref,
             wrel, wroot, gb_ref, prev_ref, out_ref):
    del prev_ref
    x = (x_ref[...].reshape(_G, _SP, _D) + pos_ref[...][None]).reshape(_ROWS, _D)
    ws = (w0, w1, w2, w3, w4)
    for i in range(5):
        x = _nt(x, ws[i][...]) + pb_ref[i, :][None, :]
        if i < 4:
            x = x * (0.5 + 0.5 * jnp.tanh(0.5 * x))
    xt = _cap(x)
    xtb = xt.astype(jnp.bfloat16)
    msgs = []
    for s in range(_G):
        a = (edge_ref[s] != 0).astype(jnp.bfloat16)
        xs = xtb[s * _SP:(s + 1) * _SP]
        msgs.append(lax.dot_general(a, xs, (((0,), (0,)), ((), ())),
                                    preferred_element_type=jnp.float32))
    msg = jnp.concatenate(msgs, axis=0)
    out_t = _nt(msg, wrel[...]) + _nt(xt, wroot[...]) + gb_ref[0, :][None, :]
    t = _cap(out_t)
    t = jnp.where(t >= 0, t, 0.01 * t)
    res = _cap(t)
    out_ref[...] = res.reshape(_G, _SP, _D)[:, :_S, :]


def _tc_dense_chunk(q, x_q, edge_pad, pos_pad, ws, pb, wrel, wroot, gb, prev):
    n_blocks = _NROWS_Q // _ROWS
    const2 = pl.BlockSpec((_D, _D), lambda i: (0, 0))
    in_specs = [
        pl.BlockSpec((_ROWS, _D), lambda i: (i, 0)),
        pl.BlockSpec((_G, _SP, _SP),
                     lambda i, q=q: (q * n_blocks + i, 0, 0)),
        pl.BlockSpec((_SP, _D), lambda i: (0, 0)),
        const2, const2, const2, const2, const2,
        pl.BlockSpec((5, _D), lambda i: (0, 0)),
        const2, const2,
        pl.BlockSpec((1, _D), lambda i: (0, 0)),
    ]
    operands = [x_q, edge_pad, pos_pad,
                ws[0], ws[1], ws[2], ws[3], ws[4],
                pb, wrel, wroot, gb]
    aliases = {}
    if prev is not None:
        in_specs.append(pl.BlockSpec(memory_space=pl.ANY))
        operands.append(prev)
        aliases = {12: 0}
    grid_spec = pl.GridSpec(
        grid=(n_blocks,),
        in_specs=in_specs,
        out_specs=pl.BlockSpec((_G, _S, _D),
                               lambda i, q=q: (q * n_blocks + i, 0, 0)),
    )
    if prev is not None:
        body = _tc_body
    else:
        def body(*refs):
            return _tc_body(*refs[:-1], None, refs[-1])
    return pl.pallas_call(
        body,
        grid_spec=grid_spec,
        out_shape=jax.ShapeDtypeStruct((_B, _S, _D), jnp.float32),
        input_output_aliases=aliases,
    )(*operands)


def kernel(params, tokens, edge):
    tokens = tokens.astype(jnp.int32)
    idx = jnp.pad(tokens, ((0, 0), (0, _SP - _S)))
    idx = idx.reshape(_NQ, _NROWS_Q)

    pos_pad = jnp.pad(params["pos_table"], ((0, _SP - _S), (0, 0)))
    edge_pad = jnp.pad(edge.astype(jnp.int32),
                       ((0, 0), (0, _SP - _S), (0, _SP - _S)))
    pb = jnp.stack(params["proj_b"])
    gb = params["gcn_b"][3][None, :]
    ws = params["proj_W"]
    wrel, wroot = params["gcn_Wrel"][3], params["gcn_Wroot"][3]

    xs = [_sc_gather(params["token_table"], idx[q]) for q in range(_NQ)]

    out = None
    for q in range(_NQ):
        out = _tc_dense_chunk(q, xs[q], edge_pad, pos_pad,
                              ws, pb, wrel, wroot, gb, out)
    return out


# speedup vs baseline: 1.2462x; 1.2462x over previous
import functools

import jax
import jax.numpy as jnp
from jax import lax
from jax.experimental import pallas as pl
from jax.experimental.pallas import tpu as pltpu
from jax.experimental.pallas import tpu_sc as plsc

_B, _S, _D, _V = 128, 77, 256, 50000
_SP = 80
_G = 8
_ROWS = _G * _SP
_NQ = 2
_BQ = _B // _NQ
_NROWS_Q = _BQ * _SP


def _sc_gather(table, idx_q):
    info = plsc.get_sparse_core_info()
    nw = info.num_cores * info.num_subcores
    b_per_w = _NROWS_Q // nw
    n_streams = b_per_w // _SP

    mesh = plsc.VectorSubcoreMesh(core_axis_name="c", subcore_axis_name="s")

    @functools.partial(
        pl.kernel,
        out_type=jax.ShapeDtypeStruct((_NROWS_Q, _D), jnp.float32),
        mesh=mesh,
        scratch_types=[
            pltpu.VMEM((n_streams, _SP), jnp.int32),
            pltpu.VMEM((b_per_w, _D), jnp.float32),
            [pltpu.SemaphoreType.DMA] * 4,
            [pltpu.SemaphoreType.DMA] * 4,
        ],
    )
    def k(table_hbm, idx_hbm, out_hbm, idx_v, rows_v, gsems, wsems):
        wid = lax.axis_index("s") * info.num_cores + lax.axis_index("c")
        base = wid * b_per_w
        pltpu.sync_copy(idx_hbm.at[wid], idx_v)
        gathers = []
        for c in range(n_streams):
            gathers.append(pltpu.async_copy(
                table_hbm.at[idx_v.at[c]],
                rows_v.at[pl.ds(c * _SP, _SP)], gsems[c % 4]))
        writes = []
        for c in range(n_streams):
            gathers[c].wait()
            writes.append(pltpu.async_copy(
                rows_v.at[pl.ds(c * _SP, _SP)],
                out_hbm.at[pl.ds(base + c * _SP, _SP)], wsems[c % 4]))
        for wcp in writes:
            wcp.wait()

    return k(table, idx_q.reshape(nw, n_streams, _SP))


def _cap(u):
    n = jnp.maximum(jnp.sqrt(jnp.sum(u * u, axis=-1, keepdims=True)), 1e-15)
    thc = jnp.minimum(jnp.tanh(n), 1.0 - 1e-7)
    s = 0.5 * jnp.log((1.0 + thc) / (1.0 - thc)) / n
    return u * s


def _nt(a, w):
    return lax.dot_general(a.astype(jnp.bfloat16), w.astype(jnp.bfloat16),
                           (((1,), (1,)), ((), ())),
                           preferred_element_type=jnp.float32)


def _tc_body(x_ref, edge_ref, pos_ref,
             w0, w1, w2, w3, w4, pb_ref,
             wrel, wroot, gb_ref, prev_ref, out_ref):
    del prev_ref
    x = (x_ref[...].reshape(_G, _SP, _D) + pos_ref[...][None]).reshape(_ROWS, _D)
    ws = (w0, w1, w2, w3, w4)
    for i in range(5):
        x = _nt(x, ws[i][...]) + pb_ref[i, :][None, :]
        if i < 4:
            x = x * (0.5 + 0.5 * jnp.tanh(0.5 * x))
    xt = _cap(x)
    xtb = xt.astype(jnp.bfloat16)
    msgs = []
    for s in range(_G):
        a = (edge_ref[s] != 0).astype(jnp.bfloat16)
        xs = xtb[s * _SP:(s + 1) * _SP]
        msgs.append(lax.dot_general(a, xs, (((0,), (0,)), ((), ())),
                                    preferred_element_type=jnp.float32))
    msg = jnp.concatenate(msgs, axis=0)
    out_t = _nt(msg, wrel[...]) + _nt(xt, wroot[...]) + gb_ref[0, :][None, :]
    t = _cap(out_t)
    t = jnp.where(t >= 0, t, 0.01 * t)
    res = _cap(t)
    res3 = jnp.transpose(res.reshape(_G, _SP, _D), (1, 0, 2))
    out_ref[...] = res3[:_S]


def _tc_dense_chunk(q, x_q, edge_pad, pos_pad, ws, pb, wrel, wroot, gb, prev):
    n_blocks = _NROWS_Q // _ROWS
    const2 = pl.BlockSpec((_D, _D), lambda i: (0, 0))
    in_specs = [
        pl.BlockSpec((_ROWS, _D), lambda i: (i, 0)),
        pl.BlockSpec((_G, _SP, _SP),
                     lambda i, q=q: (q * n_blocks + i, 0, 0)),
        pl.BlockSpec((_SP, _D), lambda i: (0, 0)),
        const2, const2, const2, const2, const2,
        pl.BlockSpec((5, _D), lambda i: (0, 0)),
        const2, const2,
        pl.BlockSpec((1, _D), lambda i: (0, 0)),
    ]
    operands = [x_q, edge_pad, pos_pad,
                ws[0], ws[1], ws[2], ws[3], ws[4],
                pb, wrel, wroot, gb]
    aliases = {}
    if prev is not None:
        in_specs.append(pl.BlockSpec(memory_space=pl.ANY))
        operands.append(prev)
        aliases = {12: 0}
    grid_spec = pl.GridSpec(
        grid=(n_blocks,),
        in_specs=in_specs,
        out_specs=pl.BlockSpec((_S, _G, _D),
                               lambda i, q=q: (0, q * n_blocks + i, 0)),
    )
    if prev is not None:
        body = _tc_body
    else:
        def body(*refs):
            return _tc_body(*refs[:-1], None, refs[-1])
    return pl.pallas_call(
        body,
        grid_spec=grid_spec,
        out_shape=jax.ShapeDtypeStruct((_S, _B, _D), jnp.float32),
        input_output_aliases=aliases,
    )(*operands)


def kernel(params, tokens, edge):
    tokens = tokens.astype(jnp.int32)
    idx = jnp.pad(tokens, ((0, 0), (0, _SP - _S)))
    idx = idx.reshape(_NQ, _NROWS_Q)

    pos_pad = jnp.pad(params["pos_table"], ((0, _SP - _S), (0, 0)))
    edge_pad = jnp.pad(edge.astype(jnp.int32),
                       ((0, 0), (0, _SP - _S), (0, _SP - _S)))
    pb = jnp.stack(params["proj_b"])
    gb = params["gcn_b"][3][None, :]
    ws = params["proj_W"]
    wrel, wroot = params["gcn_Wrel"][3], params["gcn_Wroot"][3]

    xs = [_sc_gather(params["token_table"], idx[q]) for q in range(_NQ)]

    out = None
    for q in range(_NQ):
        out = _tc_dense_chunk(q, xs[q], edge_pad, pos_pad,
                              ws, pb, wrel, wroot, gb, out)
    return jnp.swapaxes(out, 0, 1)
